# baseline (speedup 1.0000x reference)
import functools

import jax
import jax.numpy as jnp
from jax import lax
from jax.experimental import pallas as pl
from jax.experimental.pallas import tpu as pltpu
from jax.experimental.pallas import tpu_sc as plsc

_B, _S, _D = 16, 2048, 512
_SLOTS = 4
_ROWS = _B * _SLOTS


@functools.partial(
    pl.kernel,
    out_type=jax.ShapeDtypeStruct((_B, _SLOTS * _D), jnp.float32),
    mesh=plsc.VectorSubcoreMesh(core_axis_name="c", subcore_axis_name="s",
                                num_cores=1),
    scratch_types=[
        pltpu.VMEM((16,), jnp.int32),
        pltpu.VMEM((16,), jnp.int32),
        pltpu.VMEM((16,), jnp.int32),
        pltpu.VMEM((24, _D), jnp.float32),
        pltpu.SemaphoreType.DMA,
        pltpu.SemaphoreType.DMA,
    ],
)
def _encode_sc(ctx_hbm, sb_hbm, pad_hbm, out_hbm,
               sl_v, bl_v, tok_v, rows_v, sem0, sem1):
    wid = lax.axis_index("s")

    @pl.when(wid < _SLOTS)
    def _():
        j = wid
        cp_sl = pltpu.async_copy(sb_hbm.at[pl.ds(2 * _B * _S, 16)],
                                 sl_v, sem0)
        cp_bl = pltpu.async_copy(sb_hbm.at[pl.ds(2 * _B * _S + 16, 16)],
                                 bl_v, sem0)
        cp_pad = pltpu.async_copy(pad_hbm, rows_v.at[16], sem1)
        cp_sl.wait()
        cp_bl.wait()
        lane = lax.iota(jnp.int32, 16)
        is_buf = j == _SLOTS - 1
        length = jnp.where(is_buf, bl_v[...], sl_v[...])
        pos = length + jnp.where(is_buf, -1, j - 3)
        idx = (lane + jnp.where(is_buf, _B, 0)) * _S + jnp.maximum(pos, 0)
        pltpu.async_copy(sb_hbm.at[idx], tok_v, sem0).wait()

        row_idx = lane * _S + tok_v[...]
        pltpu.async_copy(ctx_hbm.at[row_idx], rows_v.at[pl.ds(0, 16)],
                         sem0).wait()
        cp_pad.wait()
        ocps = [pltpu.async_copy(rows_v.at[jnp.where(pos[b] < 0, 16, b)],
                                 out_hbm.at[b, pl.ds(j * _D, _D)], sem1)
                for b in range(_B)]
        for c in ocps:
            c.wait()


def kernel(contextualized_input_batch, stacks, buffers, stack_lengths,
           buffer_lengths, padding):
    ctx = contextualized_input_batch.reshape(_B * _S, _D)
    sb = jnp.concatenate(
        [stacks.astype(jnp.int32).reshape(_B * _S),
         buffers.astype(jnp.int32).reshape(_B * _S),
         stack_lengths.astype(jnp.int32),
         buffer_lengths.astype(jnp.int32)], axis=0)
    return _encode_sc(ctx, sb, padding)


# speedup vs baseline: 1.1262x; 1.1262x over previous
import functools

import jax
import jax.numpy as jnp
from jax import lax
from jax.experimental import pallas as pl
from jax.experimental.pallas import tpu as pltpu
from jax.experimental.pallas import tpu_sc as plsc

_B, _S, _D = 16, 2048, 512
_SLOTS = 4
_ROWS = _B * _SLOTS


@functools.partial(
    pl.kernel,
    out_type=jax.ShapeDtypeStruct((_B, _SLOTS * _D), jnp.float32),
    mesh=plsc.VectorSubcoreMesh(core_axis_name="c", subcore_axis_name="s",
                                num_cores=1),
    scratch_types=[
        pltpu.VMEM((16,), jnp.int32),
        pltpu.VMEM((16,), jnp.int32),
        pltpu.VMEM((16,), jnp.int32),
        pltpu.VMEM((24, _D), jnp.float32),
        pltpu.SemaphoreType.DMA,
        pltpu.SemaphoreType.DMA,
    ],
)
def _encode_sc(ctx_hbm, sb_hbm, sl_hbm, bl_hbm, pad_hbm, out_hbm,
               sl_v, bl_v, tok_v, rows_v, sem0, sem1):
    wid = lax.axis_index("s")

    @pl.when(wid < _SLOTS)
    def _():
        j = wid
        cp_sl = pltpu.async_copy(sl_hbm, sl_v, sem0)
        cp_bl = pltpu.async_copy(bl_hbm, bl_v, sem0)
        cp_pad = pltpu.async_copy(pad_hbm, rows_v.at[16], sem1)
        cp_sl.wait()
        cp_bl.wait()
        lane = lax.iota(jnp.int32, 16)
        is_buf = j == _SLOTS - 1
        length = jnp.where(is_buf, bl_v[...], sl_v[...])
        pos = length + jnp.where(is_buf, -1, j - 3)
        idx = (lane + jnp.where(is_buf, _B, 0)) * _S + jnp.maximum(pos, 0)
        pltpu.async_copy(sb_hbm.at[idx], tok_v, sem0).wait()

        row_idx = lane * _S + tok_v[...]
        pltpu.async_copy(ctx_hbm.at[row_idx], rows_v.at[pl.ds(0, 16)],
                         sem0).wait()
        cp_pad.wait()
        ocps = [pltpu.async_copy(rows_v.at[jnp.where(pos[b] < 0, 16, b)],
                                 out_hbm.at[b, pl.ds(j * _D, _D)], sem1)
                for b in range(_B)]
        for c in ocps:
            c.wait()


def kernel(contextualized_input_batch, stacks, buffers, stack_lengths,
           buffer_lengths, padding):
    ctx = contextualized_input_batch.reshape(_B * _S, _D)
    sb = jnp.concatenate(
        [stacks.astype(jnp.int32), buffers.astype(jnp.int32)], axis=0
    ).reshape(2 * _B * _S)
    sl = stack_lengths.astype(jnp.int32)
    bl = buffer_lengths.astype(jnp.int32)
    return _encode_sc(ctx, sb, sl, bl, padding)


# speedup vs baseline: 1.1423x; 1.0143x over previous
import functools

import jax
import jax.numpy as jnp
from jax import lax
from jax.experimental import pallas as pl
from jax.experimental.pallas import tpu as pltpu
from jax.experimental.pallas import tpu_sc as plsc

_B, _S, _D = 16, 2048, 512
_SLOTS = 4
_ROWS = _B * _SLOTS


@functools.partial(
    pl.kernel,
    out_type=jax.ShapeDtypeStruct((_B, _SLOTS * _D), jnp.float32),
    mesh=plsc.VectorSubcoreMesh(core_axis_name="c", subcore_axis_name="s",
                                num_cores=1),
    scratch_types=[
        pltpu.VMEM((16,), jnp.int32),
        pltpu.VMEM((16,), jnp.int32),
        pltpu.VMEM((16,), jnp.int32),
        pltpu.VMEM((16,), jnp.int32),
        pltpu.VMEM((16, _D), jnp.float32),
        pltpu.SemaphoreType.DMA,
        pltpu.SemaphoreType.DMA,
    ],
)
def _encode_sc(ctx_hbm, sb_hbm, sl_hbm, bl_hbm, pad_hbm, out_hbm,
               sl_v, bl_v, tok_v, ridx_v, rows_v, sem0, sem1):
    wid = lax.axis_index("s")
    j = wid % _SLOTS
    q = wid // _SLOTS
    cp_sl = pltpu.async_copy(sl_hbm, sl_v, sem0)
    cp_bl = pltpu.async_copy(bl_hbm, bl_v, sem0)
    cp_pad = pltpu.async_copy(pad_hbm, rows_v.at[8], sem1)
    cp_sl.wait()
    cp_bl.wait()
    lane = lax.iota(jnp.int32, 16)
    is_buf = j == _SLOTS - 1
    length = jnp.where(is_buf, bl_v[...], sl_v[...])
    pos = length + jnp.where(is_buf, -1, j - 3)
    idx = (lane + jnp.where(is_buf, _B, 0)) * _S + jnp.maximum(pos, 0)
    pltpu.async_copy(sb_hbm.at[idx], tok_v, sem0).wait()

    sel = _SLOTS * q + lax.rem(lane, _SLOTS)
    posr = lax.gather(
        pos, sel[:, None],
        lax.GatherDimensionNumbers(offset_dims=(), collapsed_slice_dims=(0,),
                                   start_index_map=(0,)),
        (1,), mode=lax.GatherScatterMode.PROMISE_IN_BOUNDS)
    tokr = lax.gather(
        tok_v[...], sel[:, None],
        lax.GatherDimensionNumbers(offset_dims=(), collapsed_slice_dims=(0,),
                                   start_index_map=(0,)),
        (1,), mode=lax.GatherScatterMode.PROMISE_IN_BOUNDS)
    ridx_v[...] = (_SLOTS * q + lax.rem(lane, _SLOTS)) * _S + tokr
    pltpu.async_copy(ctx_hbm.at[ridx_v.at[pl.ds(0, 8)]],
                     rows_v.at[pl.ds(0, 8)], sem0).wait()
    cp_pad.wait()
    ocps = [pltpu.async_copy(rows_v.at[jnp.where(posr[i] < 0, 8, i)],
                             out_hbm.at[_SLOTS * q + i, pl.ds(j * _D, _D)],
                             sem1)
            for i in range(_SLOTS)]
    for c in ocps:
        c.wait()


def kernel(contextualized_input_batch, stacks, buffers, stack_lengths,
           buffer_lengths, padding):
    ctx = contextualized_input_batch.reshape(_B * _S, _D)
    sb = jnp.concatenate(
        [stacks.astype(jnp.int32), buffers.astype(jnp.int32)], axis=0
    ).reshape(2 * _B * _S)
    sl = stack_lengths.astype(jnp.int32)
    bl = buffer_lengths.astype(jnp.int32)
    return _encode_sc(ctx, sb, sl, bl, padding)
